# baseline (speedup 1.0000x reference)
import functools

import jax
import jax.numpy as jnp
from jax import lax
from jax.experimental import pallas as pl
from jax.experimental.pallas import tpu as pltpu
from jax.experimental.pallas import tpu_sc as plsc

EMBEDDING_DIM = 64
PADDED_DIM = 128
NUM_CORES = 2
NUM_SUBCORES = 16
NUM_WORKERS = NUM_CORES * NUM_SUBCORES
NBUF = 2
CHUNK = 400
ROW_BLK = 1024
BB = 1024


def _pad_transpose(wt):
    dim, vocab = wt.shape

    def body(wt_ref, o_ref):
        blk = wt_ref[...]
        o_ref[...] = jnp.concatenate(
            [blk.T, jnp.zeros((ROW_BLK, PADDED_DIM - dim), jnp.float32)], axis=1
        )

    return pl.pallas_call(
        body,
        grid=(pl.cdiv(vocab, ROW_BLK),),
        in_specs=[pl.BlockSpec((dim, ROW_BLK), lambda i: (0, i))],
        out_specs=pl.BlockSpec((ROW_BLK, PADDED_DIM), lambda i: (i, 0)),
        out_shape=jax.ShapeDtypeStruct((vocab, PADDED_DIM), jnp.float32),
    )(wt)


def _sc_gather(w128, idx):
    n = idx.shape[0]
    per_worker = n // NUM_WORKERS
    n_chunks = per_worker // CHUNK
    mesh = plsc.VectorSubcoreMesh(core_axis_name="c", subcore_axis_name="s")

    @functools.partial(
        pl.kernel,
        mesh=mesh,
        compiler_params=pltpu.CompilerParams(use_tc_tiling_on_sc=False),
        out_type=jax.ShapeDtypeStruct((n, PADDED_DIM), jnp.float32),
        scratch_types=[
            pltpu.VMEM((per_worker,), jnp.int32),
        ]
        + [pltpu.VMEM((CHUNK, PADDED_DIM), jnp.float32) for _ in range(NBUF)]
        + [pltpu.SemaphoreType.DMA for _ in range(2 * NBUF)],
    )
    def gather_k(table_hbm, idx_hbm, out_hbm, idx_v, *scratch):
        bufs = scratch[:NBUF]
        gsems = scratch[NBUF : 2 * NBUF]
        wsems = scratch[2 * NBUF :]
        wid = lax.axis_index("s") * NUM_CORES + lax.axis_index("c")
        base = wid * per_worker
        pltpu.sync_copy(idx_hbm.at[pl.ds(base, per_worker)], idx_v)

        def start_gather(c):
            b = c % NBUF
            return pltpu.async_copy(
                table_hbm.at[idx_v.at[pl.ds(c * CHUNK, CHUNK)]], bufs[b], gsems[b]
            )

        gh = [None] * NBUF
        wr = [None] * NBUF
        for c in range(NBUF - 1):
            gh[c % NBUF] = start_gather(c)
        for c in range(n_chunks):
            b = c % NBUF
            nxt = c + NBUF - 1
            if nxt < n_chunks:
                nb = nxt % NBUF
                if wr[nb] is not None:
                    wr[nb].wait()
                gh[nb] = start_gather(nxt)
            gh[b].wait()
            wr[b] = pltpu.async_copy(
                bufs[b], out_hbm.at[pl.ds(base + c * CHUNK, CHUNK)], wsems[b]
            )
        for w in wr:
            if w is not None:
                w.wait()

    return gather_k(w128, idx)


def _to_batch_minor(rows, seq, batch):
    def body(in_ref, o_ref):
        o_ref[0] = in_ref[0][:, :EMBEDDING_DIM].T

    return pl.pallas_call(
        body,
        grid=(seq, batch // BB),
        in_specs=[pl.BlockSpec((1, BB, PADDED_DIM), lambda s, j: (s, j, 0))],
        out_specs=pl.BlockSpec((1, EMBEDDING_DIM, BB), lambda s, j: (s, 0, j)),
        out_shape=jax.ShapeDtypeStruct((seq, EMBEDDING_DIM, batch), jnp.float32),
    )(rows)


def kernel(x, weight):
    batch, seq = x.shape
    n = batch * seq
    idx = x.T.reshape(n)
    w128 = _pad_transpose(weight.T)
    rows = _sc_gather(w128, idx).reshape(seq, batch, PADDED_DIM)
    p = _to_batch_minor(rows, seq, batch)
    return jnp.transpose(p, (2, 0, 1))
